# baseline (speedup 1.0000x reference)
import functools

import jax
import jax.numpy as jnp
from jax import lax
from jax.experimental import pallas as pl
from jax.experimental.pallas import tpu as pltpu
from jax.experimental.pallas import tpu_sc as plsc

F32 = jnp.float32
I32 = jnp.int32

T = 2048
D = 768
F = 1024
E = 16
TM = 512
NT = T // TM
NS = 2 * NT + E
BLK = 256
RB = T // 32


def _router_body(x_ref, gw_ref, bias_ref, idx_ref, pos_ref, rw_ref,
                 counts_ref, stats_ref, meta_ref):
    x = x_ref[...]
    gw = gw_ref[...]
    bias = bias_ref[...]
    logits = lax.dot_general(x, gw, (((1,), (1,)), ((), ())),
                             preferred_element_type=F32) + bias
    m = jnp.max(logits, axis=1, keepdims=True)
    iota_e = lax.broadcasted_iota(I32, (T, E), 1)
    idx = jnp.min(jnp.where(logits == m, iota_e, E), axis=1, keepdims=True)
    idx_ref[...] = idx
    rw_ref[...] = jnp.ones((T, 1), F32)

    onehot = (idx == lax.broadcasted_iota(I32, (T, E), 1)).astype(F32)
    counts = jnp.sum(onehot, axis=0, keepdims=True)
    counts_ref[...] = counts

    meanc = jnp.sum(counts) / E
    stdc = jnp.sqrt(jnp.sum((counts - meanc) ** 2) / (E - 1))
    lb = stdc / (meanc + 1e-6)
    sl = lax.broadcasted_iota(I32, (1, E), 1)
    stats_ref[...] = jnp.where(
        sl == 0, lb, jnp.where(sl == 1, jnp.max(counts),
                               jnp.where(sl == 2, jnp.min(counts), 0.0)))

    ir = lax.broadcasted_iota(I32, (E, E), 0)
    ic = lax.broadcasted_iota(I32, (E, E), 1)
    tri_incl = (ir <= ic).astype(F32)
    ends_f = jnp.dot(counts, tri_incl, preferred_element_type=F32)
    starts_f = ends_f - counts

    nb = T // BLK
    br = lax.broadcasted_iota(I32, (BLK, BLK), 0)
    bc = lax.broadcasted_iota(I32, (BLK, BLK), 1)
    tri_strict = (br > bc).astype(F32)

    def blk_body(bi, run):
        idx_b = idx_ref[pl.ds(bi * BLK, BLK), :]
        oh = (idx_b == lax.broadcasted_iota(I32, (BLK, E), 1)).astype(F32)
        csum = jnp.dot(tri_strict, oh, preferred_element_type=F32)
        posb = jnp.sum(oh * (csum + run + starts_f), axis=1, keepdims=True)
        pos_ref[pl.ds(bi * BLK, BLK), :] = posb.astype(I32)
        return run + jnp.sum(oh, axis=0, keepdims=True)

    lax.fori_loop(0, nb, blk_body, jnp.zeros((1, E), F32))

    ends_row = ends_f
    tile_lo = (lax.broadcasted_iota(I32, (NT, 1), 0) * TM).astype(F32)
    fe = jnp.sum((ends_row <= tile_lo).astype(F32), axis=1, keepdims=True)
    le = jnp.sum((ends_row <= tile_lo + (TM - 1)).astype(F32), axis=1,
                 keepdims=True)
    n_t = le - fe + 1.0
    steps_t = n_t + 1.0
    tr = lax.broadcasted_iota(I32, (NT, NT), 0)
    tc = lax.broadcasted_iota(I32, (NT, NT), 1)
    tri_nt = (tr < tc).astype(F32)
    steps_row = jnp.reshape(steps_t, (1, NT))
    ss_row = jnp.dot(steps_row, tri_nt, preferred_element_type=F32)
    total = jnp.sum(steps_row)

    s_col = lax.broadcasted_iota(I32, (NS, 1), 0).astype(F32)
    ss_end_row = ss_row + steps_row
    t_s = jnp.sum((ss_end_row <= s_col).astype(F32), axis=1, keepdims=True)
    t_s = jnp.minimum(t_s, float(NT - 1))
    oh_t = (jnp.reshape(lax.broadcasted_iota(I32, (1, NT), 1), (1, NT))
            .astype(F32) == t_s).astype(F32)
    ss_s = jnp.sum(oh_t * ss_row, axis=1, keepdims=True)
    fe_s = jnp.sum(oh_t * jnp.reshape(fe, (1, NT)), axis=1, keepdims=True)
    le_s = jnp.sum(oh_t * jnp.reshape(le, (1, NT)), axis=1, keepdims=True)
    n_s = jnp.sum(oh_t * jnp.reshape(n_t, (1, NT)), axis=1, keepdims=True)
    o_s = s_col - ss_s
    valid = s_col < total
    routed = valid & (o_s < n_s)
    shared = valid & (o_s == n_s)
    e_s = jnp.clip(fe_s + o_s, 0.0, float(E - 1))
    le_last = jnp.max(le)
    rw_s = jnp.where(routed, e_s, jnp.where(shared, le_s, le_last))
    rw_s = jnp.clip(rw_s, 0.0, float(E - 1))
    oh_e = (jnp.reshape(lax.broadcasted_iota(I32, (1, E), 1), (1, E))
            .astype(F32) == e_s).astype(F32)
    gstart = jnp.sum(oh_e * starts_f, axis=1, keepdims=True)
    gend = jnp.sum(oh_e * ends_row, axis=1, keepdims=True)
    lo = t_s * TM
    hi = lo + TM
    m_start = jnp.where(routed, jnp.maximum(gstart, lo),
                        jnp.where(shared, lo, 0.0))
    m_end = jnp.where(routed, jnp.minimum(gend, hi),
                      jnp.where(shared, hi, 0.0))
    first = ((s_col == ss_s) & valid).astype(F32)

    sr = lax.broadcasted_iota(I32, (NS, NS), 0)
    sc2 = lax.broadcasted_iota(I32, (NS, NS), 1)
    shift = (sr == sc2 + 1).astype(F32)
    prev_rw = jnp.dot(shift, rw_s, preferred_element_type=F32)
    new_rw = ((s_col == 0.0) | (rw_s != prev_rw)).astype(F32)
    tril_ns = (sr >= sc2).astype(F32)
    cum_shared = jnp.dot(tril_ns, shared.astype(F32),
                         preferred_element_type=F32)
    shared_first = (shared & (cum_shared == 1.0)).astype(F32)

    rows = [t_s, rw_s, m_start, m_end, shared.astype(F32), first,
            new_rw, shared_first]
    meta = jnp.concatenate([jnp.reshape(r, (1, NS)) for r in rows], axis=0)
    meta_ref[...] = meta.astype(I32)


def _router(xf, gate_w, expert_bias):
    out_shapes = (
        jax.ShapeDtypeStruct((T, 1), I32),
        jax.ShapeDtypeStruct((T, 1), I32),
        jax.ShapeDtypeStruct((T, 1), F32),
        jax.ShapeDtypeStruct((1, E), F32),
        jax.ShapeDtypeStruct((1, E), F32),
        jax.ShapeDtypeStruct((8, NS), I32),
    )
    return pl.pallas_call(_router_body, out_shape=out_shapes)(
        xf, gate_w, expert_bias.reshape(1, E))


BF16 = jnp.bfloat16


def _ffn_tile(xw, wg, wu, wd):
    xb = xw.astype(BF16)
    g = lax.dot_general(xb, wg, (((1,), (1,)), ((), ())),
                        preferred_element_type=F32)
    u = lax.dot_general(xb, wu, (((1,), (1,)), ((), ())),
                        preferred_element_type=F32)
    h = (jax.nn.silu(g) * u).astype(BF16)
    return lax.dot_general(h, wd, (((1,), (1,)), ((), ())),
                           preferred_element_type=F32)


def _gmm_body(meta_ref, xs_ref, rwg_ref, rwu_ref, rwd_ref,
              swg_ref, swu_ref, swd_ref, out_ref,
              wgb_ref, wub_ref, wdb_ref, sgb_ref, sub_ref, sdb_ref):
    s = pl.program_id(0)
    t = meta_ref[0, s]
    start = meta_ref[2, s]
    end = meta_ref[3, s]
    is_shared = meta_ref[4, s] == 1
    first = meta_ref[5, s] == 1

    @pl.when(meta_ref[6, s] == 1)
    def _convert_routed():
        wgb_ref[...] = rwg_ref[0].astype(BF16)
        wub_ref[...] = rwu_ref[0].astype(BF16)
        wdb_ref[...] = rwd_ref[0].astype(BF16)

    @pl.when(meta_ref[7, s] == 1)
    def _convert_shared():
        sgb_ref[...] = swg_ref[...].astype(BF16)
        sub_ref[...] = swu_ref[...].astype(BF16)
        sdb_ref[...] = swd_ref[...].astype(BF16)

    @pl.when(first)
    def _zero():
        out_ref[...] = jnp.zeros_like(out_ref)

    rows = t * TM + lax.broadcasted_iota(I32, (TM, 1), 0)
    active = (rows >= start) & (rows < end)
    nonempty = end > start

    @pl.when(nonempty & jnp.logical_not(is_shared))
    def _routed():
        xw = jnp.where(active, xs_ref[...], 0.0)
        out_ref[...] += _ffn_tile(xw, wgb_ref[...], wub_ref[...], wdb_ref[...])

    @pl.when(nonempty & is_shared)
    def _shared():
        xw = jnp.where(active, xs_ref[...], 0.0)
        out_ref[...] += _ffn_tile(xw, sgb_ref[...], sub_ref[...], sdb_ref[...])


def _gmm(meta, xs, rwg, rwu, rwd, swg, swu, swd):
    grid_spec = pltpu.PrefetchScalarGridSpec(
        num_scalar_prefetch=1,
        grid=(NS,),
        in_specs=[
            pl.BlockSpec((TM, D), lambda s, m: (m[0, s], 0)),
            pl.BlockSpec((1, F, D), lambda s, m: (m[1, s], 0, 0)),
            pl.BlockSpec((1, F, D), lambda s, m: (m[1, s], 0, 0)),
            pl.BlockSpec((1, D, F), lambda s, m: (m[1, s], 0, 0)),
            pl.BlockSpec((F, D), lambda s, m: (0, 0)),
            pl.BlockSpec((F, D), lambda s, m: (0, 0)),
            pl.BlockSpec((D, F), lambda s, m: (0, 0)),
        ],
        out_specs=pl.BlockSpec((TM, D), lambda s, m: (m[0, s], 0)),
        scratch_shapes=[
            pltpu.VMEM((F, D), BF16),
            pltpu.VMEM((F, D), BF16),
            pltpu.VMEM((D, F), BF16),
            pltpu.VMEM((F, D), BF16),
            pltpu.VMEM((F, D), BF16),
            pltpu.VMEM((D, F), BF16),
        ],
    )
    return pl.pallas_call(
        _gmm_body,
        grid_spec=grid_spec,
        out_shape=jax.ShapeDtypeStruct((T, D), F32),
        compiler_params=pltpu.CompilerParams(
            dimension_semantics=("arbitrary",)),
    )(meta, xs, rwg, rwu, rwd, swg, swu, swd)


def _make_sc_info():
    info = plsc.get_sparse_core_info()
    return info.num_cores, info.num_subcores


def _dispatch(xf, pos):
    nc, ns = _make_sc_info()
    mesh = plsc.VectorSubcoreMesh(core_axis_name="c", subcore_axis_name="s")

    @functools.partial(
        pl.kernel, mesh=mesh,
        out_type=jax.ShapeDtypeStruct((T, D), F32),
        scratch_types=[
            pltpu.VMEM((T,), I32),
            pltpu.VMEM((T,), I32),
            pltpu.VMEM((RB,), I32),
            pltpu.VMEM((RB, D), F32),
            pltpu.SemaphoreType.DMA,
        ],
        compiler_params=pltpu.CompilerParams(needs_layout_passes=False),
    )
    def d_kernel(x_hbm, pos_hbm, xs_hbm, pos_v, perm_v, myidx_v, rows_v, sem):
        wid = lax.axis_index("s") * nc + lax.axis_index("c")
        base = wid * RB
        pltpu.sync_copy(pos_hbm, pos_v)

        def body(i, carry):
            pc = pos_v[pl.ds(i * 16, 16)]
            vals = lax.iota(I32, 16) + i * 16
            plsc.store_scatter(perm_v, [pc], vals)
            return carry

        lax.fori_loop(0, T // 16, body, 0)

        def copy_body(j, carry):
            myidx_v[pl.ds(j * 16, 16)] = perm_v[pl.ds(base + j * 16, 16)]
            return carry

        lax.fori_loop(0, RB // 16, copy_body, 0)
        pltpu.async_copy(x_hbm.at[myidx_v], rows_v, sem).wait()
        pltpu.sync_copy(rows_v, xs_hbm.at[pl.ds(base, RB)])

    return d_kernel(xf, pos)


def _unsort(ys, pos):
    nc, ns = _make_sc_info()
    mesh = plsc.VectorSubcoreMesh(core_axis_name="c", subcore_axis_name="s")

    @functools.partial(
        pl.kernel, mesh=mesh,
        out_type=jax.ShapeDtypeStruct((T, D), F32),
        scratch_types=[
            pltpu.VMEM((RB,), I32),
            pltpu.VMEM((RB, D), F32),
            pltpu.SemaphoreType.DMA,
        ],
    )
    def u_kernel(ys_hbm, pos_hbm, out_hbm, idx_v, rows_v, sem):
        wid = lax.axis_index("s") * nc + lax.axis_index("c")
        base = wid * RB
        pltpu.sync_copy(pos_hbm.at[pl.ds(base, RB)], idx_v)
        pltpu.async_copy(ys_hbm.at[idx_v], rows_v, sem).wait()
        pltpu.sync_copy(rows_v, out_hbm.at[pl.ds(base, RB)])

    return u_kernel(ys, pos)


def kernel(x, gate_w, expert_bias, shared_gate_w, shared_up_w, shared_down_w,
           routed_gate_w, routed_up_w, routed_down_w):
    b, s, d = x.shape
    xf = x.reshape(T, D)

    idx, pos, rw, counts, stats, meta = _router(xf, gate_w, expert_bias)
    pos1 = pos.reshape(T)

    xs = _dispatch(xf, pos1)
    ys = _gmm(meta, xs, routed_gate_w, routed_up_w, routed_down_w,
              shared_gate_w, shared_up_w, shared_down_w)
    out = _unsort(ys, pos1).reshape(b, s, d)

    counts_v = counts.reshape(E)
    load_balance = stats[0, 0]
    cmax = stats[0, 1]
    cmin = stats[0, 2]
    return (out,
            rw.reshape(b, s, 1),
            idx.reshape(b, s, 1),
            counts_v,
            load_balance,
            cmax,
            cmin)


# speedup vs baseline: 1.0238x; 1.0238x over previous
import functools

import jax
import jax.numpy as jnp
from jax import lax
from jax.experimental import pallas as pl
from jax.experimental.pallas import tpu as pltpu
from jax.experimental.pallas import tpu_sc as plsc

F32 = jnp.float32
I32 = jnp.int32

T = 2048
D = 768
F = 1024
E = 16
TM = 256
NT = T // TM
NS = 2 * NT + E
BLK = 256
RB = T // 32


def _router_body(x_ref, gw_ref, bias_ref, idx_ref, pos_ref, rw_ref,
                 counts_ref, stats_ref, meta_ref):
    x = x_ref[...]
    gw = gw_ref[...]
    bias = bias_ref[...]
    logits = lax.dot_general(x, gw, (((1,), (1,)), ((), ())),
                             preferred_element_type=F32) + bias
    m = jnp.max(logits, axis=1, keepdims=True)
    iota_e = lax.broadcasted_iota(I32, (T, E), 1)
    idx = jnp.min(jnp.where(logits == m, iota_e, E), axis=1, keepdims=True)
    idx_ref[...] = idx
    rw_ref[...] = jnp.ones((T, 1), F32)

    onehot = (idx == lax.broadcasted_iota(I32, (T, E), 1)).astype(F32)
    counts = jnp.sum(onehot, axis=0, keepdims=True)
    counts_ref[...] = counts

    meanc = jnp.sum(counts) / E
    stdc = jnp.sqrt(jnp.sum((counts - meanc) ** 2) / (E - 1))
    lb = stdc / (meanc + 1e-6)
    sl = lax.broadcasted_iota(I32, (1, E), 1)
    stats_ref[...] = jnp.where(
        sl == 0, lb, jnp.where(sl == 1, jnp.max(counts),
                               jnp.where(sl == 2, jnp.min(counts), 0.0)))

    ir = lax.broadcasted_iota(I32, (E, E), 0)
    ic = lax.broadcasted_iota(I32, (E, E), 1)
    tri_incl = (ir <= ic).astype(F32)
    ends_f = jnp.dot(counts, tri_incl, preferred_element_type=F32)
    starts_f = ends_f - counts

    nb = T // BLK
    br = lax.broadcasted_iota(I32, (BLK, BLK), 0)
    bc = lax.broadcasted_iota(I32, (BLK, BLK), 1)
    tri_strict = (br > bc).astype(F32)

    def blk_body(bi, run):
        idx_b = idx_ref[pl.ds(bi * BLK, BLK), :]
        oh = (idx_b == lax.broadcasted_iota(I32, (BLK, E), 1)).astype(F32)
        csum = jnp.dot(tri_strict, oh, preferred_element_type=F32)
        posb = jnp.sum(oh * (csum + run + starts_f), axis=1, keepdims=True)
        pos_ref[pl.ds(bi * BLK, BLK), :] = posb.astype(I32)
        return run + jnp.sum(oh, axis=0, keepdims=True)

    lax.fori_loop(0, nb, blk_body, jnp.zeros((1, E), F32))

    ends_row = ends_f
    tile_lo = (lax.broadcasted_iota(I32, (NT, 1), 0) * TM).astype(F32)
    fe = jnp.sum((ends_row <= tile_lo).astype(F32), axis=1, keepdims=True)
    le = jnp.sum((ends_row <= tile_lo + (TM - 1)).astype(F32), axis=1,
                 keepdims=True)
    n_t = le - fe + 1.0
    steps_t = n_t + 1.0
    tr = lax.broadcasted_iota(I32, (NT, NT), 0)
    tc = lax.broadcasted_iota(I32, (NT, NT), 1)
    tri_nt = (tr < tc).astype(F32)
    steps_row = jnp.reshape(steps_t, (1, NT))
    ss_row = jnp.dot(steps_row, tri_nt, preferred_element_type=F32)
    total = jnp.sum(steps_row)

    s_col = lax.broadcasted_iota(I32, (NS, 1), 0).astype(F32)
    ss_end_row = ss_row + steps_row
    t_s = jnp.sum((ss_end_row <= s_col).astype(F32), axis=1, keepdims=True)
    t_s = jnp.minimum(t_s, float(NT - 1))
    oh_t = (jnp.reshape(lax.broadcasted_iota(I32, (1, NT), 1), (1, NT))
            .astype(F32) == t_s).astype(F32)
    ss_s = jnp.sum(oh_t * ss_row, axis=1, keepdims=True)
    fe_s = jnp.sum(oh_t * jnp.reshape(fe, (1, NT)), axis=1, keepdims=True)
    le_s = jnp.sum(oh_t * jnp.reshape(le, (1, NT)), axis=1, keepdims=True)
    n_s = jnp.sum(oh_t * jnp.reshape(n_t, (1, NT)), axis=1, keepdims=True)
    o_s = s_col - ss_s
    valid = s_col < total
    routed = valid & (o_s < n_s)
    shared = valid & (o_s == n_s)
    e_s = jnp.clip(fe_s + o_s, 0.0, float(E - 1))
    le_last = jnp.max(le)
    rw_s = jnp.where(routed, e_s, jnp.where(shared, le_s, le_last))
    rw_s = jnp.clip(rw_s, 0.0, float(E - 1))
    oh_e = (jnp.reshape(lax.broadcasted_iota(I32, (1, E), 1), (1, E))
            .astype(F32) == e_s).astype(F32)
    gstart = jnp.sum(oh_e * starts_f, axis=1, keepdims=True)
    gend = jnp.sum(oh_e * ends_row, axis=1, keepdims=True)
    lo = t_s * TM
    hi = lo + TM
    m_start = jnp.where(routed, jnp.maximum(gstart, lo),
                        jnp.where(shared, lo, 0.0))
    m_end = jnp.where(routed, jnp.minimum(gend, hi),
                      jnp.where(shared, hi, 0.0))
    first = ((s_col == ss_s) & valid).astype(F32)

    sr = lax.broadcasted_iota(I32, (NS, NS), 0)
    sc2 = lax.broadcasted_iota(I32, (NS, NS), 1)
    shift = (sr == sc2 + 1).astype(F32)
    prev_rw = jnp.dot(shift, rw_s, preferred_element_type=F32)
    new_rw = ((s_col == 0.0) | (rw_s != prev_rw)).astype(F32)
    tril_ns = (sr >= sc2).astype(F32)
    cum_shared = jnp.dot(tril_ns, shared.astype(F32),
                         preferred_element_type=F32)
    shared_first = (shared & (cum_shared == 1.0)).astype(F32)

    rows = [t_s, rw_s, m_start, m_end, shared.astype(F32), first,
            new_rw, shared_first]
    meta = jnp.concatenate([jnp.reshape(r, (1, NS)) for r in rows], axis=0)
    meta_ref[...] = meta.astype(I32)


def _router(xf, gate_w, expert_bias):
    out_shapes = (
        jax.ShapeDtypeStruct((T, 1), I32),
        jax.ShapeDtypeStruct((T, 1), I32),
        jax.ShapeDtypeStruct((T, 1), F32),
        jax.ShapeDtypeStruct((1, E), F32),
        jax.ShapeDtypeStruct((1, E), F32),
        jax.ShapeDtypeStruct((8, NS), I32),
    )
    return pl.pallas_call(_router_body, out_shape=out_shapes)(
        xf, gate_w, expert_bias.reshape(1, E))


BF16 = jnp.bfloat16


def _ffn_tile(xw, wg, wu, wd):
    xb = xw.astype(BF16)
    g = lax.dot_general(xb, wg, (((1,), (1,)), ((), ())),
                        preferred_element_type=F32)
    u = lax.dot_general(xb, wu, (((1,), (1,)), ((), ())),
                        preferred_element_type=F32)
    h = (jax.nn.silu(g) * u).astype(BF16)
    return lax.dot_general(h, wd, (((1,), (1,)), ((), ())),
                           preferred_element_type=F32)


def _gmm_body(meta_ref, xs_ref, rwg_ref, rwu_ref, rwd_ref,
              swg_ref, swu_ref, swd_ref, out_ref,
              wgb_ref, wub_ref, wdb_ref, sgb_ref, sub_ref, sdb_ref):
    s = pl.program_id(0)
    t = meta_ref[0, s]
    start = meta_ref[2, s]
    end = meta_ref[3, s]
    is_shared = meta_ref[4, s] == 1
    first = meta_ref[5, s] == 1

    @pl.when(meta_ref[6, s] == 1)
    def _convert_routed():
        wgb_ref[...] = rwg_ref[0].astype(BF16)
        wub_ref[...] = rwu_ref[0].astype(BF16)
        wdb_ref[...] = rwd_ref[0].astype(BF16)

    @pl.when(meta_ref[7, s] == 1)
    def _convert_shared():
        sgb_ref[...] = swg_ref[...].astype(BF16)
        sub_ref[...] = swu_ref[...].astype(BF16)
        sdb_ref[...] = swd_ref[...].astype(BF16)

    @pl.when(first)
    def _zero():
        out_ref[...] = jnp.zeros_like(out_ref)

    rows = t * TM + lax.broadcasted_iota(I32, (TM, 1), 0)
    active = (rows >= start) & (rows < end)
    nonempty = end > start

    @pl.when(nonempty & jnp.logical_not(is_shared))
    def _routed():
        xw = jnp.where(active, xs_ref[...], 0.0)
        out_ref[...] += _ffn_tile(xw, wgb_ref[...], wub_ref[...], wdb_ref[...])

    @pl.when(nonempty & is_shared)
    def _shared():
        xw = jnp.where(active, xs_ref[...], 0.0)
        out_ref[...] += _ffn_tile(xw, sgb_ref[...], sub_ref[...], sdb_ref[...])


def _gmm(meta, xs, rwg, rwu, rwd, swg, swu, swd):
    grid_spec = pltpu.PrefetchScalarGridSpec(
        num_scalar_prefetch=1,
        grid=(NS,),
        in_specs=[
            pl.BlockSpec((TM, D), lambda s, m: (m[0, s], 0)),
            pl.BlockSpec((1, F, D), lambda s, m: (m[1, s], 0, 0)),
            pl.BlockSpec((1, F, D), lambda s, m: (m[1, s], 0, 0)),
            pl.BlockSpec((1, D, F), lambda s, m: (m[1, s], 0, 0)),
            pl.BlockSpec((F, D), lambda s, m: (0, 0)),
            pl.BlockSpec((F, D), lambda s, m: (0, 0)),
            pl.BlockSpec((D, F), lambda s, m: (0, 0)),
        ],
        out_specs=pl.BlockSpec((TM, D), lambda s, m: (m[0, s], 0)),
        scratch_shapes=[
            pltpu.VMEM((F, D), BF16),
            pltpu.VMEM((F, D), BF16),
            pltpu.VMEM((D, F), BF16),
            pltpu.VMEM((F, D), BF16),
            pltpu.VMEM((F, D), BF16),
            pltpu.VMEM((D, F), BF16),
        ],
    )
    return pl.pallas_call(
        _gmm_body,
        grid_spec=grid_spec,
        out_shape=jax.ShapeDtypeStruct((T, D), F32),
        compiler_params=pltpu.CompilerParams(
            dimension_semantics=("arbitrary",)),
    )(meta, xs, rwg, rwu, rwd, swg, swu, swd)


def _make_sc_info():
    info = plsc.get_sparse_core_info()
    return info.num_cores, info.num_subcores


def _dispatch(xf, pos):
    nc, ns = _make_sc_info()
    mesh = plsc.VectorSubcoreMesh(core_axis_name="c", subcore_axis_name="s")

    @functools.partial(
        pl.kernel, mesh=mesh,
        out_type=jax.ShapeDtypeStruct((T, D), F32),
        scratch_types=[
            pltpu.VMEM((T,), I32),
            pltpu.VMEM((T,), I32),
            pltpu.VMEM((RB,), I32),
            pltpu.VMEM((RB, D), F32),
            pltpu.SemaphoreType.DMA,
        ],
        compiler_params=pltpu.CompilerParams(needs_layout_passes=False),
    )
    def d_kernel(x_hbm, pos_hbm, xs_hbm, pos_v, perm_v, myidx_v, rows_v, sem):
        wid = lax.axis_index("s") * nc + lax.axis_index("c")
        base = wid * RB
        pltpu.sync_copy(pos_hbm, pos_v)

        def body(i, carry):
            pc = pos_v[pl.ds(i * 16, 16)]
            vals = lax.iota(I32, 16) + i * 16
            plsc.store_scatter(perm_v, [pc], vals)
            return carry

        lax.fori_loop(0, T // 16, body, 0)

        def copy_body(j, carry):
            myidx_v[pl.ds(j * 16, 16)] = perm_v[pl.ds(base + j * 16, 16)]
            return carry

        lax.fori_loop(0, RB // 16, copy_body, 0)
        pltpu.async_copy(x_hbm.at[myidx_v], rows_v, sem).wait()
        pltpu.sync_copy(rows_v, xs_hbm.at[pl.ds(base, RB)])

    return d_kernel(xf, pos)


def _unsort(ys, pos):
    nc, ns = _make_sc_info()
    mesh = plsc.VectorSubcoreMesh(core_axis_name="c", subcore_axis_name="s")

    @functools.partial(
        pl.kernel, mesh=mesh,
        out_type=jax.ShapeDtypeStruct((T, D), F32),
        scratch_types=[
            pltpu.VMEM((RB,), I32),
            pltpu.VMEM((RB, D), F32),
            pltpu.SemaphoreType.DMA,
        ],
    )
    def u_kernel(ys_hbm, pos_hbm, out_hbm, idx_v, rows_v, sem):
        wid = lax.axis_index("s") * nc + lax.axis_index("c")
        base = wid * RB
        pltpu.sync_copy(pos_hbm.at[pl.ds(base, RB)], idx_v)
        pltpu.async_copy(ys_hbm.at[idx_v], rows_v, sem).wait()
        pltpu.sync_copy(rows_v, out_hbm.at[pl.ds(base, RB)])

    return u_kernel(ys, pos)


def kernel(x, gate_w, expert_bias, shared_gate_w, shared_up_w, shared_down_w,
           routed_gate_w, routed_up_w, routed_down_w):
    b, s, d = x.shape
    xf = x.reshape(T, D)

    idx, pos, rw, counts, stats, meta = _router(xf, gate_w, expert_bias)
    pos1 = pos.reshape(T)

    xs = _dispatch(xf, pos1)
    ys = _gmm(meta, xs, routed_gate_w, routed_up_w, routed_down_w,
              shared_gate_w, shared_up_w, shared_down_w)
    out = _unsort(ys, pos1).reshape(b, s, d)

    counts_v = counts.reshape(E)
    load_balance = stats[0, 0]
    cmax = stats[0, 1]
    cmin = stats[0, 2]
    return (out,
            rw.reshape(b, s, 1),
            idx.reshape(b, s, 1),
            counts_v,
            load_balance,
            cmax,
            cmin)
